# Initial kernel scaffold; baseline (speedup 1.0000x reference)
import jax
import jax.numpy as jnp
from jax.experimental import pallas as pl


def kernel(x, adj_indices, adj_values, W1, b1, W2, b2):
    raise NotImplementedError("write your pallas kernel here")


# speedup vs baseline: 4.6213x; 4.6213x over previous
import functools

import jax
import jax.numpy as jnp
from jax import lax
from jax.experimental import pallas as pl
from jax.experimental.pallas import tpu as pltpu
from jax.experimental.pallas import tpu_sc as plsc

NC = 2
NS = 16
L = 16

BN = 2000


def _tc_support1(x, W1):
    N, F = x.shape
    D, _, H = W1.shape

    def body(x_ref, w_ref, o_ref):
        o_ref[0] = jnp.dot(x_ref[...], w_ref[0],
                           preferred_element_type=jnp.float32)

    return pl.pallas_call(
        body,
        grid=(N // BN, D),
        in_specs=[
            pl.BlockSpec((BN, F), lambda jn, i: (jn, 0)),
            pl.BlockSpec((1, F, H), lambda jn, i: (i, 0, 0)),
        ],
        out_specs=pl.BlockSpec((1, BN, H), lambda jn, i: (i, jn, 0)),
        out_shape=jax.ShapeDtypeStruct((D, N, H), jnp.float32),
    )(x, W1)


def _tc_layer2(agg1, b1, W2):
    D, N, H = agg1.shape
    C = W2.shape[2]
    DH = D * H

    def body(a_ref, b_ref, w_ref, o_ref, h_s):
        i = pl.program_id(1)

        @pl.when(i == 0)
        def _():
            for i2 in range(D):
                h_s[:, i2 * H:(i2 + 1) * H] = jnp.maximum(
                    a_ref[i2] + b_ref[i2][None, :], 0.0)

        o_ref[0] = jnp.dot(h_s[...], w_ref[0],
                           preferred_element_type=jnp.float32)

    return pl.pallas_call(
        body,
        grid=(N // BN, D),
        in_specs=[
            pl.BlockSpec((D, BN, H), lambda jn, i: (0, jn, 0)),
            pl.BlockSpec((D, H), lambda jn, i: (0, 0)),
            pl.BlockSpec((1, DH, C), lambda jn, i: (i, 0, 0)),
        ],
        out_specs=pl.BlockSpec((1, BN, C), lambda jn, i: (i, jn, 0)),
        out_shape=jax.ShapeDtypeStruct((D, N, C), jnp.float32),
        scratch_shapes=[pltpu.VMEM((BN, DH), jnp.float32)],
    )(agg1, b1, W2)


def _tc_pool(agg2, b2):
    D, N, C = agg2.shape

    def body(a_ref, b_ref, o_ref):
        a = jnp.maximum(a_ref[...] + b_ref[...][:, None, :], 0.0)
        o_ref[...] = jnp.max(a, axis=0)

    return pl.pallas_call(
        body,
        grid=(N // BN,),
        in_specs=[
            pl.BlockSpec((D, BN, C), lambda jn: (0, jn, 0)),
            pl.BlockSpec((D, C), lambda jn: (0, 0)),
        ],
        out_specs=pl.BlockSpec((BN, C), lambda jn: (jn, 0)),
        out_shape=jax.ShapeDtypeStruct((N, C), jnp.float32),
    )(agg2, b2)


def _sc_spmm(adj_rows, adj_gcols, adj_vals, sup_flat, D, N, E, W):
    CH = 128
    NCHUNK = E // CH
    TPT = (NCHUNK + NS - 1) // NS
    KMAX = (D + NC - 1) // NC
    ZR = ((N // NS) + 7) // 8 * 8
    ZLAST = N - ZR * (NS - 1)

    mesh = plsc.VectorSubcoreMesh(core_axis_name="c", subcore_axis_name="s")

    scratch = [
        pltpu.VMEM((CH,), jnp.int32),
        pltpu.VMEM((CH,), jnp.int32),
        pltpu.VMEM((CH,), jnp.float32),
        pltpu.VMEM((CH, W), jnp.float32),
        pltpu.VMEM((ZR, W), jnp.float32),
        pltpu.VMEM_SHARED((N, W), jnp.float32),
        pltpu.SemaphoreType.DMA,
    ]

    @functools.partial(
        pl.kernel,
        out_type=jax.ShapeDtypeStruct((D * N, W), jnp.float32),
        mesh=mesh,
        scratch_types=scratch,
        compiler_params=pltpu.CompilerParams(use_tc_tiling_on_sc=False),
    )
    def k(row_hbm, col_hbm, val_hbm, sup_hbm, out_hbm,
          col_v, row_v, val_v, gath_v, zero_v, acc_sh, sem):
        c = lax.axis_index("c")
        s = lax.axis_index("s")

        def zb(j, _):
            zero_v[j, :] = jnp.zeros((W,), jnp.float32)
            return 0
        lax.fori_loop(0, ZR, zb, 0)

        def zero_acc():
            @pl.when(s < NS - 1)
            def _():
                pltpu.sync_copy(zero_v, acc_sh.at[pl.ds(s * ZR, ZR), :])

            @pl.when(s == NS - 1)
            def _():
                pltpu.sync_copy(zero_v.at[pl.ds(0, ZLAST), :],
                                acc_sh.at[pl.ds((NS - 1) * ZR, ZLAST), :])

        def writeback(i):
            base = i * N

            @pl.when(s < NS - 1)
            def _():
                pltpu.sync_copy(acc_sh.at[pl.ds(s * ZR, ZR), :],
                                out_hbm.at[pl.ds(base + s * ZR, ZR), :])

            @pl.when(s == NS - 1)
            def _():
                pltpu.sync_copy(
                    acc_sh.at[pl.ds((NS - 1) * ZR, ZLAST), :],
                    out_hbm.at[pl.ds(base + (NS - 1) * ZR, ZLAST), :])

        def process(i, e0):
            pltpu.sync_copy(col_hbm.at[i, pl.ds(e0, CH)], col_v)
            pltpu.sync_copy(row_hbm.at[i, pl.ds(e0, CH)], row_v)
            pltpu.sync_copy(val_hbm.at[i, pl.ds(e0, CH)], val_v)
            pltpu.async_copy(sup_hbm.at[col_v], gath_v, sem).wait()

            def mul(g, _):
                v16 = val_v[pl.ds(g * L, L)]
                base = g * L
                for lane in range(L):
                    splat = jnp.broadcast_to(v16[lane], (L,))
                    gath_v[base + lane, :] = gath_v[base + lane, :] * splat
                return 0
            lax.fori_loop(0, CH // L, mul, 0)
            pltpu.sync_copy(gath_v, acc_sh.at[row_v], add=True)

        zero_acc()
        plsc.subcore_barrier()

        def adj_body(kk, _):
            i = kk * NC + c

            @pl.when(i < D)
            def _():
                def chunk(t, _c):
                    j = s + t * NS

                    @pl.when(j < NCHUNK)
                    def _():
                        process(i, j * CH)
                    return 0
                lax.fori_loop(0, TPT, chunk, 0)
                plsc.subcore_barrier()
                writeback(i)
                zero_acc()
                plsc.subcore_barrier()
            return 0

        lax.fori_loop(0, KMAX, adj_body, 0)

    return k(adj_rows, adj_gcols, adj_vals, sup_flat)


def kernel(x, adj_indices, adj_values, W1, b1, W2, b2):
    N, F = x.shape
    D, _, E = adj_indices.shape
    H = W1.shape[2]
    C = W2.shape[2]

    adj_idx = adj_indices.astype(jnp.int32)
    rows = adj_idx[:, 0, :]
    gcols = adj_idx[:, 1, :] + (jnp.arange(D, dtype=jnp.int32) * N)[:, None]

    sup1 = _tc_support1(x, W1)
    agg1 = _sc_spmm(rows, gcols, adj_values, sup1.reshape(D * N, H),
                    D, N, E, H)
    sup2 = _tc_layer2(agg1.reshape(D, N, H), b1, W2)
    agg2 = _sc_spmm(rows, gcols, adj_values, sup2.reshape(D * N, C),
                    D, N, E, C)
    return _tc_pool(agg2.reshape(D, N, C), b2)


# speedup vs baseline: 10.9463x; 2.3687x over previous
import functools

import jax
import jax.numpy as jnp
from jax import lax
from jax.experimental import pallas as pl
from jax.experimental.pallas import tpu as pltpu
from jax.experimental.pallas import tpu_sc as plsc

NC = 2
NS = 16
L = 16

BN = 2000


def _tc_support1(x, W1):
    N, F = x.shape
    D, _, H = W1.shape

    def body(x_ref, w_ref, o_ref):
        o_ref[0] = jnp.dot(x_ref[...], w_ref[0],
                           preferred_element_type=jnp.float32)

    return pl.pallas_call(
        body,
        grid=(N // BN, D),
        in_specs=[
            pl.BlockSpec((BN, F), lambda jn, i: (jn, 0)),
            pl.BlockSpec((1, F, H), lambda jn, i: (i, 0, 0)),
        ],
        out_specs=pl.BlockSpec((1, BN, H), lambda jn, i: (i, jn, 0)),
        out_shape=jax.ShapeDtypeStruct((D, N, H), jnp.float32),
    )(x, W1)


def _tc_layer2(agg1, b1, W2):
    D, N, H = agg1.shape
    C = W2.shape[2]
    DH = D * H

    def body(a_ref, b_ref, w_ref, o_ref, h_s):
        i = pl.program_id(1)

        @pl.when(i == 0)
        def _():
            for i2 in range(D):
                h_s[:, i2 * H:(i2 + 1) * H] = jnp.maximum(
                    a_ref[i2] + b_ref[i2][None, :], 0.0)

        o_ref[0] = jnp.dot(h_s[...], w_ref[0],
                           preferred_element_type=jnp.float32)

    return pl.pallas_call(
        body,
        grid=(N // BN, D),
        in_specs=[
            pl.BlockSpec((D, BN, H), lambda jn, i: (0, jn, 0)),
            pl.BlockSpec((D, H), lambda jn, i: (0, 0)),
            pl.BlockSpec((1, DH, C), lambda jn, i: (i, 0, 0)),
        ],
        out_specs=pl.BlockSpec((1, BN, C), lambda jn, i: (i, jn, 0)),
        out_shape=jax.ShapeDtypeStruct((D, N, C), jnp.float32),
        scratch_shapes=[pltpu.VMEM((BN, DH), jnp.float32)],
    )(agg1, b1, W2)


def _tc_pool(agg2, b2):
    D, N, C = agg2.shape

    def body(a_ref, b_ref, o_ref):
        a = jnp.maximum(a_ref[...] + b_ref[...][:, None, :], 0.0)
        o_ref[...] = jnp.max(a, axis=0)

    return pl.pallas_call(
        body,
        grid=(N // BN,),
        in_specs=[
            pl.BlockSpec((D, BN, C), lambda jn: (0, jn, 0)),
            pl.BlockSpec((D, C), lambda jn: (0, 0)),
        ],
        out_specs=pl.BlockSpec((BN, C), lambda jn: (jn, 0)),
        out_shape=jax.ShapeDtypeStruct((N, C), jnp.float32),
    )(agg2, b2)


def _sc_spmm(adj_rows, adj_gcols, adj_vals, sup_flat, D, N, E, W):
    CH = 128
    NCHUNK = E // CH
    CBASE_Q = NCHUNK // NS
    CEXTRA = NCHUNK - CBASE_Q * NS
    KMAX = (D + NC - 1) // NC
    ZR = ((N // NS) + 7) // 8 * 8
    ZLAST = N - ZR * (NS - 1)
    NB = 3

    mesh = plsc.VectorSubcoreMesh(core_axis_name="c", subcore_axis_name="s")

    scratch = [
        pltpu.VMEM((CBASE_Q + 1, CH), jnp.int32),
        pltpu.VMEM((CBASE_Q + 1, CH), jnp.int32),
        pltpu.VMEM((CBASE_Q + 1, CH), jnp.float32),
        [pltpu.VMEM((CH, W), jnp.float32) for _ in range(NB)],
        pltpu.VMEM((ZR, W), jnp.float32),
        pltpu.VMEM_SHARED((N, W), jnp.float32),
        [pltpu.SemaphoreType.DMA for _ in range(NB)],
        [pltpu.SemaphoreType.DMA for _ in range(NB)],
    ]

    @functools.partial(
        pl.kernel,
        out_type=jax.ShapeDtypeStruct((D * N, W), jnp.float32),
        mesh=mesh,
        scratch_types=scratch,
        compiler_params=pltpu.CompilerParams(use_tc_tiling_on_sc=False),
    )
    def k(row_hbm, col_hbm, val_hbm, sup_hbm, out_hbm,
          col_v, row_v, val_v, gath, zero_v, acc_sh, sg, ss):
        c = lax.axis_index("c")
        s = lax.axis_index("s")

        def zb(j, _):
            zero_v[j, :] = jnp.zeros((W,), jnp.float32)
            return 0
        lax.fori_loop(0, ZR, zb, 0)

        def zero_acc():
            @pl.when(s < NS - 1)
            def _():
                pltpu.sync_copy(zero_v, acc_sh.at[pl.ds(s * ZR, ZR), :])

            @pl.when(s == NS - 1)
            def _():
                pltpu.sync_copy(zero_v.at[pl.ds(0, ZLAST), :],
                                acc_sh.at[pl.ds((NS - 1) * ZR, ZLAST), :])

        def writeback(i):
            base = i * N

            @pl.when(s < NS - 1)
            def _():
                pltpu.sync_copy(acc_sh.at[pl.ds(s * ZR, ZR), :],
                                out_hbm.at[pl.ds(base + s * ZR, ZR), :])

            @pl.when(s == NS - 1)
            def _():
                pltpu.sync_copy(
                    acc_sh.at[pl.ds((NS - 1) * ZR, ZLAST), :],
                    out_hbm.at[pl.ds(base + (NS - 1) * ZR, ZLAST), :])

        cbase = s * CBASE_Q + jnp.minimum(s, CEXTRA)
        cnt = jnp.where(s < CEXTRA, CBASE_Q + 1, CBASE_Q)

        def start_gather(t, b):
            pltpu.async_copy(sup_hbm.at[col_v.at[t]], gath[b], sg[b])

        def wait_gather(b):
            pltpu.make_async_copy(sup_hbm.at[col_v.at[0]], gath[b],
                                  sg[b]).wait()

        def start_scatter(t, b):
            pltpu.async_copy(gath[b], acc_sh.at[row_v.at[t]], ss[b],
                             add=True)

        def wait_scatter(b):
            pltpu.make_async_copy(gath[b], acc_sh.at[row_v.at[0]],
                                  ss[b]).wait()

        def mul(t, b):
            def mul_g(g, _):
                v16 = val_v[t, pl.ds(g * L, L)]
                base = g * L
                for lane in range(L):
                    splat = jnp.broadcast_to(v16[lane], (L,))
                    gath[b][base + lane, :] = gath[b][base + lane, :] * splat
                return 0
            lax.fori_loop(0, CH // L, mul_g, 0)

        zero_acc()
        plsc.subcore_barrier()

        def adj_body(kk, _):
            i = kk * NC + c

            @pl.when(i < D)
            def _():
                @pl.when(s < CEXTRA)
                def _():
                    pltpu.sync_copy(
                        col_hbm.at[i, pl.ds(cbase, CBASE_Q + 1), :], col_v)
                    pltpu.sync_copy(
                        row_hbm.at[i, pl.ds(cbase, CBASE_Q + 1), :], row_v)
                    pltpu.sync_copy(
                        val_hbm.at[i, pl.ds(cbase, CBASE_Q + 1), :], val_v)

                @pl.when(s >= CEXTRA)
                def _():
                    pltpu.sync_copy(
                        col_hbm.at[i, pl.ds(cbase, CBASE_Q), :],
                        col_v.at[pl.ds(0, CBASE_Q), :])
                    pltpu.sync_copy(
                        row_hbm.at[i, pl.ds(cbase, CBASE_Q), :],
                        row_v.at[pl.ds(0, CBASE_Q), :])
                    pltpu.sync_copy(
                        val_hbm.at[i, pl.ds(cbase, CBASE_Q), :],
                        val_v.at[pl.ds(0, CBASE_Q), :])

                start_gather(0, 0)
                wait_gather(0)
                start_gather(1, 1)
                mul(0, 0)
                start_scatter(0, 0)
                wait_gather(1)
                start_gather(2, 2)
                mul(1, 1)
                start_scatter(1, 1)
                wait_gather(2)
                wait_scatter(0)
                start_gather(3, 0)
                mul(2, 2)
                start_scatter(2, 2)

                def triple(tp, _t):
                    t0 = 3 * tp
                    wait_gather(0)
                    wait_scatter(1)
                    start_gather(t0 + 1, 1)
                    mul(t0, 0)
                    start_scatter(t0, 0)
                    wait_gather(1)
                    wait_scatter(2)
                    start_gather(t0 + 2, 2)
                    mul(t0 + 1, 1)
                    start_scatter(t0 + 1, 1)
                    wait_gather(2)

                    @pl.when(t0 + 3 < cnt)
                    def _():
                        wait_scatter(0)
                        start_gather(t0 + 3, 0)
                    mul(t0 + 2, 2)
                    start_scatter(t0 + 2, 2)
                    return 0
                lax.fori_loop(1, CBASE_Q // 3, triple, 0)

                @pl.when(s < CEXTRA)
                def _():
                    wait_gather(0)
                    mul(CBASE_Q, 0)
                    start_scatter(CBASE_Q, 0)

                wait_scatter(0)
                wait_scatter(1)
                wait_scatter(2)
                plsc.subcore_barrier()
                writeback(i)
                zero_acc()
                plsc.subcore_barrier()
            return 0

        lax.fori_loop(0, KMAX, adj_body, 0)

    return k(adj_rows.reshape(D, NCHUNK, CH),
             adj_gcols.reshape(D, NCHUNK, CH),
             adj_vals.reshape(D, NCHUNK, CH),
             sup_flat)


def kernel(x, adj_indices, adj_values, W1, b1, W2, b2):
    N, F = x.shape
    D, _, E = adj_indices.shape
    H = W1.shape[2]
    C = W2.shape[2]

    adj_idx = adj_indices.astype(jnp.int32)
    rows = adj_idx[:, 0, :]
    gcols = adj_idx[:, 1, :] + (jnp.arange(D, dtype=jnp.int32) * N)[:, None]

    sup1 = _tc_support1(x, W1)
    agg1 = _sc_spmm(rows, gcols, adj_values, sup1.reshape(D * N, H),
                    D, N, E, H)
    sup2 = _tc_layer2(agg1.reshape(D, N, H), b1, W2)
    agg2 = _sc_spmm(rows, gcols, adj_values, sup2.reshape(D * N, C),
                    D, N, E, C)
    return _tc_pool(agg2.reshape(D, N, C), b2)


# speedup vs baseline: 19.4191x; 1.7740x over previous
import functools

import jax
import jax.numpy as jnp
from jax import lax
from jax.experimental import pallas as pl
from jax.experimental.pallas import tpu as pltpu
from jax.experimental.pallas import tpu_sc as plsc

NC = 2
NS = 16
L = 16

BN = 2000


def _tc_support1(x, W1):
    N, F = x.shape
    D, _, H = W1.shape

    def body(x_ref, w_ref, o_ref):
        o_ref[0] = jnp.dot(x_ref[...], w_ref[0],
                           preferred_element_type=jnp.float32)

    return pl.pallas_call(
        body,
        grid=(N // BN, D),
        in_specs=[
            pl.BlockSpec((BN, F), lambda jn, i: (jn, 0)),
            pl.BlockSpec((1, F, H), lambda jn, i: (i, 0, 0)),
        ],
        out_specs=pl.BlockSpec((1, BN, H), lambda jn, i: (i, jn, 0)),
        out_shape=jax.ShapeDtypeStruct((D, N, H), jnp.float32),
    )(x, W1)


def _tc_layer2(agg1, b1, W2):
    D, N, H = agg1.shape
    C = W2.shape[2]
    DH = D * H

    def body(a_ref, b_ref, w_ref, o_ref, h_s):
        i = pl.program_id(1)

        @pl.when(i == 0)
        def _():
            for i2 in range(D):
                h_s[:, i2 * H:(i2 + 1) * H] = jnp.maximum(
                    a_ref[i2] + b_ref[i2][None, :], 0.0)

        o_ref[0] = jnp.dot(h_s[...], w_ref[0],
                           preferred_element_type=jnp.float32)

    return pl.pallas_call(
        body,
        grid=(N // BN, D),
        in_specs=[
            pl.BlockSpec((D, BN, H), lambda jn, i: (0, jn, 0)),
            pl.BlockSpec((D, H), lambda jn, i: (0, 0)),
            pl.BlockSpec((1, DH, C), lambda jn, i: (i, 0, 0)),
        ],
        out_specs=pl.BlockSpec((1, BN, C), lambda jn, i: (i, jn, 0)),
        out_shape=jax.ShapeDtypeStruct((D, N, C), jnp.float32),
        scratch_shapes=[pltpu.VMEM((BN, DH), jnp.float32)],
    )(agg1, b1, W2)


def _tc_pool(agg2, b2):
    D, N, C = agg2.shape

    def body(a_ref, b_ref, o_ref):
        a = jnp.maximum(a_ref[...] + b_ref[...][:, None, :], 0.0)
        o_ref[...] = jnp.max(a, axis=0)

    return pl.pallas_call(
        body,
        grid=(N // BN,),
        in_specs=[
            pl.BlockSpec((D, BN, C), lambda jn: (0, jn, 0)),
            pl.BlockSpec((D, C), lambda jn: (0, 0)),
        ],
        out_specs=pl.BlockSpec((BN, C), lambda jn: (jn, 0)),
        out_shape=jax.ShapeDtypeStruct((N, C), jnp.float32),
    )(agg2, b2)


def _sc_spmm(adj_rows, adj_gcols, adj_vals, sup_flat, D, N, E, W):
    CH = 128
    NCHUNK = E // CH
    CBASE_Q = NCHUNK // NS
    CEXTRA = NCHUNK - CBASE_Q * NS
    KMAX = (D + NC - 1) // NC
    ZR = ((N // NS) + 7) // 8 * 8
    ZLAST = N - ZR * (NS - 1)
    NB = 6
    PD = 3

    mesh = plsc.VectorSubcoreMesh(core_axis_name="c", subcore_axis_name="s")

    scratch = [
        pltpu.VMEM((CBASE_Q + 1, CH), jnp.int32),
        pltpu.VMEM((CBASE_Q + 1, CH), jnp.int32),
        pltpu.VMEM((CBASE_Q + 1, CH), jnp.float32),
        [pltpu.VMEM((CH, W), jnp.float32) for _ in range(NB)],
        pltpu.VMEM((ZR, W), jnp.float32),
        pltpu.VMEM_SHARED((N, W), jnp.float32),
        [pltpu.SemaphoreType.DMA for _ in range(NB)],
        [pltpu.SemaphoreType.DMA for _ in range(NB)],
    ]

    @functools.partial(
        pl.kernel,
        out_type=jax.ShapeDtypeStruct((D * N, W), jnp.float32),
        mesh=mesh,
        scratch_types=scratch,
        compiler_params=pltpu.CompilerParams(use_tc_tiling_on_sc=False),
    )
    def k(row_hbm, col_hbm, val_hbm, sup_hbm, out_hbm,
          col_v, row_v, val_v, gath, zero_v, acc_sh, sg, ss):
        c = lax.axis_index("c")
        s = lax.axis_index("s")

        def zb(j, _):
            zero_v[j, :] = jnp.zeros((W,), jnp.float32)
            return 0
        lax.fori_loop(0, ZR, zb, 0)

        def zero_acc():
            @pl.when(s < NS - 1)
            def _():
                pltpu.sync_copy(zero_v, acc_sh.at[pl.ds(s * ZR, ZR), :])

            @pl.when(s == NS - 1)
            def _():
                pltpu.sync_copy(zero_v.at[pl.ds(0, ZLAST), :],
                                acc_sh.at[pl.ds((NS - 1) * ZR, ZLAST), :])

        def writeback(i):
            base = i * N

            @pl.when(s < NS - 1)
            def _():
                pltpu.sync_copy(acc_sh.at[pl.ds(s * ZR, ZR), :],
                                out_hbm.at[pl.ds(base + s * ZR, ZR), :])

            @pl.when(s == NS - 1)
            def _():
                pltpu.sync_copy(
                    acc_sh.at[pl.ds((NS - 1) * ZR, ZLAST), :],
                    out_hbm.at[pl.ds(base + (NS - 1) * ZR, ZLAST), :])

        cbase = s * CBASE_Q + jnp.minimum(s, CEXTRA)
        cnt = jnp.where(s < CEXTRA, CBASE_Q + 1, CBASE_Q)

        def start_gather(t, b):
            pltpu.async_copy(sup_hbm.at[col_v.at[t]], gath[b], sg[b])

        def wait_gather(b):
            pltpu.make_async_copy(sup_hbm.at[col_v.at[0]], gath[b],
                                  sg[b]).wait()

        def start_scatter(t, b):
            pltpu.async_copy(gath[b], acc_sh.at[row_v.at[t]], ss[b],
                             add=True)

        def wait_scatter(b):
            pltpu.make_async_copy(gath[b], acc_sh.at[row_v.at[0]],
                                  ss[b]).wait()

        def mul(t, b):
            def mul_g(g, _):
                v16 = val_v[t, pl.ds(g * L, L)]
                base = g * L
                for lane in range(L):
                    splat = jnp.broadcast_to(v16[lane], (L,))
                    gath[b][base + lane, :] = gath[b][base + lane, :] * splat
                return 0
            lax.fori_loop(0, CH // L, mul_g, 0, unroll=2)

        zero_acc()
        plsc.subcore_barrier()

        def adj_body(kk, _):
            i = kk * NC + c

            @pl.when(i < D)
            def _():
                @pl.when(s < CEXTRA)
                def _():
                    a = pltpu.async_copy(
                        col_hbm.at[i, pl.ds(cbase, CBASE_Q + 1), :], col_v,
                        ss[0])
                    b = pltpu.async_copy(
                        row_hbm.at[i, pl.ds(cbase, CBASE_Q + 1), :], row_v,
                        ss[0])
                    d = pltpu.async_copy(
                        val_hbm.at[i, pl.ds(cbase, CBASE_Q + 1), :], val_v,
                        ss[0])
                    a.wait(); b.wait(); d.wait()

                @pl.when(s >= CEXTRA)
                def _():
                    a = pltpu.async_copy(
                        col_hbm.at[i, pl.ds(cbase, CBASE_Q), :],
                        col_v.at[pl.ds(0, CBASE_Q), :], ss[0])
                    b = pltpu.async_copy(
                        row_hbm.at[i, pl.ds(cbase, CBASE_Q), :],
                        row_v.at[pl.ds(0, CBASE_Q), :], ss[0])
                    d = pltpu.async_copy(
                        val_hbm.at[i, pl.ds(cbase, CBASE_Q), :],
                        val_v.at[pl.ds(0, CBASE_Q), :], ss[0])
                    a.wait(); b.wait(); d.wait()

                for t in range(PD):
                    start_gather(t, t)

                def slot(t, b, wait_sc, guard):
                    wait_gather(b)
                    bp = (b + PD) % NB

                    def prefetch():
                        if wait_sc:
                            wait_scatter(bp)
                        start_gather(t + PD, bp)
                    if guard:
                        pl.when(t + PD < cnt)(prefetch)
                    else:
                        prefetch()
                    mul(t, b)
                    start_scatter(t, b)

                for t in range(PD):
                    slot(t, t, False, False)
                for t in range(PD, NB):
                    slot(t, t, True, False)

                def sextet(tp, _t):
                    t0 = NB * tp
                    for b in range(NB):
                        slot(t0 + b, b, True, True)
                    return 0
                lax.fori_loop(1, CBASE_Q // NB, sextet, 0)

                @pl.when(s < CEXTRA)
                def _():
                    wait_gather(CBASE_Q % NB)
                    mul(CBASE_Q, CBASE_Q % NB)
                    start_scatter(CBASE_Q, CBASE_Q % NB)

                for b in range(NB):
                    wait_scatter(b)
                plsc.subcore_barrier()
                writeback(i)
                zero_acc()
                plsc.subcore_barrier()
            return 0

        lax.fori_loop(0, KMAX, adj_body, 0)

    return k(adj_rows.reshape(D, NCHUNK, CH),
             adj_gcols.reshape(D, NCHUNK, CH),
             adj_vals.reshape(D, NCHUNK, CH),
             sup_flat)


def kernel(x, adj_indices, adj_values, W1, b1, W2, b2):
    N, F = x.shape
    D, _, E = adj_indices.shape
    H = W1.shape[2]
    C = W2.shape[2]

    adj_idx = adj_indices.astype(jnp.int32)
    rows = adj_idx[:, 0, :]
    gcols = adj_idx[:, 1, :] + (jnp.arange(D, dtype=jnp.int32) * N)[:, None]

    sup1 = _tc_support1(x, W1)
    agg1 = _sc_spmm(rows, gcols, adj_values, sup1.reshape(D * N, H),
                    D, N, E, H)
    sup2 = _tc_layer2(agg1.reshape(D, N, H), b1, W2)
    agg2 = _sc_spmm(rows, gcols, adj_values, sup2.reshape(D * N, C),
                    D, N, E, C)
    return _tc_pool(agg2.reshape(D, N, C), b2)


# speedup vs baseline: 24.4752x; 1.2604x over previous
import functools

import jax
import jax.numpy as jnp
from jax import lax
from jax.experimental import pallas as pl
from jax.experimental.pallas import tpu as pltpu
from jax.experimental.pallas import tpu_sc as plsc

NC = 2
NS = 16
L = 16

BN = 2000


def _tc_support1(x, W1):
    N, F = x.shape
    D, _, H = W1.shape
    NBL = N // BN

    def body(x_ref, w_ref, o_ref):
        o_ref[...] = jnp.dot(x_ref[...], w_ref[0],
                             preferred_element_type=jnp.float32)

    return pl.pallas_call(
        body,
        grid=(NBL, D),
        in_specs=[
            pl.BlockSpec((BN, F), lambda jn, i: (jn, 0)),
            pl.BlockSpec((1, F, H), lambda jn, i: (i, 0, 0)),
        ],
        out_specs=pl.BlockSpec((BN, H), lambda jn, i: (i * NBL + jn, 0)),
        out_shape=jax.ShapeDtypeStruct((D * N, H), jnp.float32),
    )(x, W1)


def _tc_layer2(h, b1flat, W2):
    N, DH = h.shape
    D, _, C = W2.shape
    NBL = N // BN

    def body(h_ref, b_ref, w_ref, o_ref, hs):
        i = pl.program_id(1)

        @pl.when(i == 0)
        def _():
            hs[...] = jnp.maximum(h_ref[...] + b_ref[...], 0.0)

        o_ref[...] = jnp.dot(hs[...], w_ref[0],
                             preferred_element_type=jnp.float32)

    return pl.pallas_call(
        body,
        grid=(NBL, D),
        in_specs=[
            pl.BlockSpec((BN, DH), lambda jn, i: (jn, 0)),
            pl.BlockSpec((1, DH), lambda jn, i: (0, 0)),
            pl.BlockSpec((1, DH, C), lambda jn, i: (i, 0, 0)),
        ],
        out_specs=pl.BlockSpec((BN, C), lambda jn, i: (i * NBL + jn, 0)),
        out_shape=jax.ShapeDtypeStruct((D * N, C), jnp.float32),
        scratch_shapes=[pltpu.VMEM((BN, DH), jnp.float32)],
    )(h, b1flat, W2)


def _tc_pool(h2, b2flat, D, C):
    N, _ = h2.shape

    def body(a_ref, b_ref, o_ref):
        a = jnp.maximum(a_ref[...] + b_ref[...], 0.0)
        m = a[:, 0:C]
        for i in range(1, D):
            m = jnp.maximum(m, a[:, i * C:(i + 1) * C])
        o_ref[...] = m

    return pl.pallas_call(
        body,
        grid=(N // BN,),
        in_specs=[
            pl.BlockSpec((BN, D * C), lambda jn: (jn, 0)),
            pl.BlockSpec((1, D * C), lambda jn: (0, 0)),
        ],
        out_specs=pl.BlockSpec((BN, C), lambda jn: (jn, 0)),
        out_shape=jax.ShapeDtypeStruct((N, C), jnp.float32),
    )(h2, b2flat)


def _sc_spmm(adj4, vals3, sup_flat, D, N, E, W):
    CH = 128
    NCHUNK = E // CH
    CBASE_Q = NCHUNK // NS
    CEXTRA = NCHUNK - CBASE_Q * NS
    KMAX = (D + NC - 1) // NC
    ZR = ((N // NS) + 7) // 8 * 8
    ZLAST = N - ZR * (NS - 1)
    NB = 6
    PD = 3

    mesh = plsc.VectorSubcoreMesh(core_axis_name="c", subcore_axis_name="s")

    scratch = [
        pltpu.VMEM((CBASE_Q + 1, CH), jnp.int32),
        pltpu.VMEM((CBASE_Q + 1, CH), jnp.int32),
        pltpu.VMEM((CBASE_Q + 1, CH), jnp.float32),
        [pltpu.VMEM((CH, W), jnp.float32) for _ in range(NB)],
        pltpu.VMEM((ZR, W), jnp.float32),
        pltpu.VMEM_SHARED((N, W), jnp.float32),
        [pltpu.SemaphoreType.DMA for _ in range(NB)],
        [pltpu.SemaphoreType.DMA for _ in range(NB)],
    ]

    @functools.partial(
        pl.kernel,
        out_type=jax.ShapeDtypeStruct((N, D * W), jnp.float32),
        mesh=mesh,
        scratch_types=scratch,
        compiler_params=pltpu.CompilerParams(use_tc_tiling_on_sc=False),
    )
    def k(adj_hbm, val_hbm, sup_hbm, out_hbm,
          col_v, row_v, val_v, gath, zero_v, acc_sh, sg, ss):
        c = lax.axis_index("c")
        s = lax.axis_index("s")

        def zb(j, _):
            zero_v[j, :] = jnp.zeros((W,), jnp.float32)
            return 0
        lax.fori_loop(0, ZR, zb, 0)

        def zero_acc():
            @pl.when(s < NS - 1)
            def _():
                pltpu.sync_copy(zero_v, acc_sh.at[pl.ds(s * ZR, ZR), :])

            @pl.when(s == NS - 1)
            def _():
                pltpu.sync_copy(zero_v.at[pl.ds(0, ZLAST), :],
                                acc_sh.at[pl.ds((NS - 1) * ZR, ZLAST), :])

        def writeback(i):
            co = i * W

            @pl.when(s < NS - 1)
            def _():
                pltpu.sync_copy(acc_sh.at[pl.ds(s * ZR, ZR), :],
                                out_hbm.at[pl.ds(s * ZR, ZR),
                                           pl.ds(co, W)])

            @pl.when(s == NS - 1)
            def _():
                pltpu.sync_copy(
                    acc_sh.at[pl.ds((NS - 1) * ZR, ZLAST), :],
                    out_hbm.at[pl.ds((NS - 1) * ZR, ZLAST), pl.ds(co, W)])

        cbase = s * CBASE_Q + jnp.minimum(s, CEXTRA)
        cnt = jnp.where(s < CEXTRA, CBASE_Q + 1, CBASE_Q)

        def offset_cols(t, ioff):
            for g in range(CH // L):
                col_v[t, pl.ds(g * L, L)] = (
                    col_v[t, pl.ds(g * L, L)] + ioff)

        def start_gather(t, b):
            pltpu.async_copy(sup_hbm.at[col_v.at[t]], gath[b], sg[b])

        def wait_gather(b):
            pltpu.make_async_copy(sup_hbm.at[col_v.at[0]], gath[b],
                                  sg[b]).wait()

        def start_scatter(t, b):
            pltpu.async_copy(gath[b], acc_sh.at[row_v.at[t]], ss[b],
                             add=True)

        def wait_scatter(b):
            pltpu.make_async_copy(gath[b], acc_sh.at[row_v.at[0]],
                                  ss[b]).wait()

        def mul(t, b):
            def mul_g(g, _):
                v16 = val_v[t, pl.ds(g * L, L)]
                base = g * L
                for lane in range(L):
                    splat = jnp.broadcast_to(v16[lane], (L,))
                    gath[b][base + lane, :] = gath[b][base + lane, :] * splat
                return 0
            lax.fori_loop(0, CH // L, mul_g, 0, unroll=2)

        zero_acc()
        plsc.subcore_barrier()

        def adj_body(kk, _):
            i = kk * NC + c
            ioff = i * N

            @pl.when(i < D)
            def _():
                @pl.when(s < CEXTRA)
                def _():
                    a = pltpu.async_copy(
                        adj_hbm.at[i, 1, pl.ds(cbase, CBASE_Q + 1), :],
                        col_v, ss[0])
                    b = pltpu.async_copy(
                        adj_hbm.at[i, 0, pl.ds(cbase, CBASE_Q + 1), :],
                        row_v, ss[0])
                    d = pltpu.async_copy(
                        val_hbm.at[i, pl.ds(cbase, CBASE_Q + 1), :],
                        val_v, ss[0])
                    a.wait(); b.wait(); d.wait()

                @pl.when(s >= CEXTRA)
                def _():
                    a = pltpu.async_copy(
                        adj_hbm.at[i, 1, pl.ds(cbase, CBASE_Q), :],
                        col_v.at[pl.ds(0, CBASE_Q), :], ss[0])
                    b = pltpu.async_copy(
                        adj_hbm.at[i, 0, pl.ds(cbase, CBASE_Q), :],
                        row_v.at[pl.ds(0, CBASE_Q), :], ss[0])
                    d = pltpu.async_copy(
                        val_hbm.at[i, pl.ds(cbase, CBASE_Q), :],
                        val_v.at[pl.ds(0, CBASE_Q), :], ss[0])
                    a.wait(); b.wait(); d.wait()

                for t in range(PD):
                    offset_cols(t, ioff)
                    start_gather(t, t)

                def slot(t, b, wait_sc, guard):
                    wait_gather(b)
                    bp = (b + PD) % NB

                    def prefetch():
                        if wait_sc:
                            wait_scatter(bp)
                        offset_cols(t + PD, ioff)
                        start_gather(t + PD, bp)
                    if guard:
                        pl.when(t + PD < cnt)(prefetch)
                    else:
                        prefetch()
                    mul(t, b)
                    start_scatter(t, b)

                for t in range(PD):
                    slot(t, t, False, False)
                for t in range(PD, NB):
                    slot(t, t, True, False)

                def sextet(tp, _t):
                    t0 = NB * tp
                    for b in range(NB):
                        slot(t0 + b, b, True, True)
                    return 0
                lax.fori_loop(1, CBASE_Q // NB, sextet, 0)

                @pl.when(s < CEXTRA)
                def _():
                    wait_gather(CBASE_Q % NB)
                    mul(CBASE_Q, CBASE_Q % NB)
                    start_scatter(CBASE_Q, CBASE_Q % NB)

                for b in range(NB):
                    wait_scatter(b)
                plsc.subcore_barrier()
                writeback(i)
                zero_acc()
                plsc.subcore_barrier()
            return 0

        lax.fori_loop(0, KMAX, adj_body, 0)

    return k(adj4, vals3, sup_flat)


def kernel(x, adj_indices, adj_values, W1, b1, W2, b2):
    N, F = x.shape
    D, _, E = adj_indices.shape
    H = W1.shape[2]
    C = W2.shape[2]
    CH = 128

    adj4 = adj_indices.astype(jnp.int32).reshape(D, 2, E // CH, CH)
    vals3 = adj_values.reshape(D, E // CH, CH)

    sup1 = _tc_support1(x, W1)
    h = _sc_spmm(adj4, vals3, sup1, D, N, E, H)
    sup2 = _tc_layer2(h, b1.reshape(1, D * H), W2)
    h2 = _sc_spmm(adj4, vals3, sup2, D, N, E, C)
    return _tc_pool(h2, b2.reshape(1, D * C), D, C)


# speedup vs baseline: 29.5487x; 1.2073x over previous
import functools

import jax
import jax.numpy as jnp
from jax import lax
from jax.experimental import pallas as pl
from jax.experimental.pallas import tpu as pltpu
from jax.experimental.pallas import tpu_sc as plsc

NC = 2
NS = 16
L = 16

BN = 2000
PADW = 512


def _tc_matmul(xin, w, relu_bias=None):
    N, K = xin.shape

    if relu_bias is None:
        def body(x_ref, w_ref, o_ref):
            o_ref[...] = jnp.dot(x_ref[...], w_ref[...],
                                 preferred_element_type=jnp.float32)
        args = (xin, w)
        in_specs = [
            pl.BlockSpec((BN, K), lambda jn: (jn, 0)),
            pl.BlockSpec((K, PADW), lambda jn: (0, 0)),
        ]
    else:
        def body(x_ref, b_ref, w_ref, o_ref):
            a = jnp.maximum(x_ref[...] + b_ref[...], 0.0)
            o_ref[...] = jnp.dot(a, w_ref[...],
                                 preferred_element_type=jnp.float32)
        args = (xin, relu_bias, w)
        in_specs = [
            pl.BlockSpec((BN, K), lambda jn: (jn, 0)),
            pl.BlockSpec((1, K), lambda jn: (0, 0)),
            pl.BlockSpec((K, PADW), lambda jn: (0, 0)),
        ]

    return pl.pallas_call(
        body,
        grid=(N // BN,),
        in_specs=in_specs,
        out_specs=pl.BlockSpec((BN, PADW), lambda jn: (jn, 0)),
        out_shape=jax.ShapeDtypeStruct((N, PADW), jnp.float32),
    )(*args)


def _tc_pool(h2, b2pad, D, C):
    N, _ = h2.shape

    def body(a_ref, b_ref, o_ref):
        a = jnp.maximum(a_ref[...] + b_ref[...], 0.0)
        m = a[:, 0:C]
        for i in range(1, D):
            m = jnp.maximum(m, a[:, i * C:(i + 1) * C])
        o_ref[...] = m

    return pl.pallas_call(
        body,
        grid=(N // BN,),
        in_specs=[
            pl.BlockSpec((BN, PADW), lambda jn: (jn, 0)),
            pl.BlockSpec((1, PADW), lambda jn: (0, 0)),
        ],
        out_specs=pl.BlockSpec((BN, C), lambda jn: (jn, 0)),
        out_shape=jax.ShapeDtypeStruct((N, C), jnp.float32),
    )(h2, b2pad)


def _sc_spmm(adj_idx, adj_vals, sup_units, D, N, E, W):
    CH = 128
    NCHUNK = E // CH
    CBASE_Q = NCHUNK // NS
    CEXTRA = NCHUNK - CBASE_Q * NS
    KMAX = (D + NC - 1) // NC
    ZR = ((N // NS) + 7) // 8 * 8
    ZLAST = N - ZR * (NS - 1)
    NB = 6
    PD = 3
    UPR = PADW // W
    EPT_MAX = (CBASE_Q + 1) * CH

    mesh = plsc.VectorSubcoreMesh(core_axis_name="c", subcore_axis_name="s")

    scratch = [
        pltpu.VMEM((EPT_MAX,), jnp.int32),
        pltpu.VMEM((EPT_MAX,), jnp.int32),
        pltpu.VMEM((EPT_MAX,), jnp.float32),
        [pltpu.VMEM((CH,), jnp.int32) for _ in range(NB)],
        [pltpu.VMEM((CH, W), jnp.float32) for _ in range(NB)],
        pltpu.VMEM((ZR, W), jnp.float32),
        pltpu.VMEM_SHARED((N, W), jnp.float32),
        [pltpu.SemaphoreType.DMA for _ in range(NB)],
        [pltpu.SemaphoreType.DMA for _ in range(NB)],
    ]

    @functools.partial(
        pl.kernel,
        out_type=jax.ShapeDtypeStruct((N, PADW), jnp.float32),
        mesh=mesh,
        scratch_types=scratch,
        compiler_params=pltpu.CompilerParams(use_tc_tiling_on_sc=False),
    )
    def k(adj_hbm, val_hbm, sup_hbm, out_hbm,
          col_v, row_v, val_v, rst, gath, zero_v, acc_sh, sg, ss):
        c = lax.axis_index("c")
        s = lax.axis_index("s")

        def zb(j, _):
            zero_v[j, :] = jnp.zeros((W,), jnp.float32)
            return 0
        lax.fori_loop(0, ZR, zb, 0)

        def zero_acc():
            @pl.when(s < NS - 1)
            def _():
                pltpu.sync_copy(zero_v, acc_sh.at[pl.ds(s * ZR, ZR), :])

            @pl.when(s == NS - 1)
            def _():
                pltpu.sync_copy(zero_v.at[pl.ds(0, ZLAST), :],
                                acc_sh.at[pl.ds((NS - 1) * ZR, ZLAST), :])

        def out_block(co, width_src):
            @pl.when(s < NS - 1)
            def _():
                pltpu.sync_copy(width_src[0],
                                out_hbm.at[pl.ds(s * ZR, ZR), pl.ds(co, W)])

            @pl.when(s == NS - 1)
            def _():
                pltpu.sync_copy(
                    width_src[1],
                    out_hbm.at[pl.ds((NS - 1) * ZR, ZLAST), pl.ds(co, W)])

        def writeback(i):
            out_block(i * W, (acc_sh.at[pl.ds(s * ZR, ZR), :],
                              acc_sh.at[pl.ds((NS - 1) * ZR, ZLAST), :]))

        for j in range(D, UPR):
            out_block(j * W, (zero_v, zero_v.at[pl.ds(0, ZLAST), :]))

        cbase = s * CBASE_Q + jnp.minimum(s, CEXTRA)
        cnt = jnp.where(s < CEXTRA, CBASE_Q + 1, CBASE_Q)

        def prep_chunk(t, b, i):
            for g in range(CH // L):
                sl = pl.ds(t * CH + g * L, L)
                col_v[sl] = col_v[sl] * UPR + i
                rst[b][pl.ds(g * L, L)] = row_v[sl]

        def start_gather(t, b):
            pltpu.async_copy(
                sup_hbm.at[col_v.at[pl.ds(t * CH, CH)]], gath[b], sg[b])

        def wait_gather(b):
            pltpu.make_async_copy(sup_hbm.at[col_v.at[pl.ds(0, CH)]],
                                  gath[b], sg[b]).wait()

        def start_scatter(b):
            pltpu.async_copy(gath[b], acc_sh.at[rst[b]], ss[b], add=True)

        def wait_scatter(b):
            pltpu.make_async_copy(gath[b], acc_sh.at[rst[b]], ss[b]).wait()

        def mul(t, b):
            def mul_g(g, _):
                v16 = val_v[pl.ds(t * CH + g * L, L)]
                base = g * L
                for lane in range(L):
                    splat = jnp.broadcast_to(v16[lane], (L,))
                    gath[b][base + lane, :] = gath[b][base + lane, :] * splat
                return 0
            lax.fori_loop(0, CH // L, mul_g, 0, unroll=2)

        zero_acc()
        plsc.subcore_barrier()

        def adj_body(kk, _):
            i = kk * NC + c

            @pl.when(i < D)
            def _():
                e0 = cbase * CH

                @pl.when(s < CEXTRA)
                def _():
                    a = pltpu.async_copy(
                        adj_hbm.at[i, 1, pl.ds(e0, EPT_MAX)],
                        col_v, ss[0])
                    b = pltpu.async_copy(
                        adj_hbm.at[i, 0, pl.ds(e0, EPT_MAX)],
                        row_v, ss[0])
                    d = pltpu.async_copy(
                        val_hbm.at[i, pl.ds(e0, EPT_MAX)],
                        val_v, ss[0])
                    a.wait(); b.wait(); d.wait()

                @pl.when(s >= CEXTRA)
                def _():
                    a = pltpu.async_copy(
                        adj_hbm.at[i, 1, pl.ds(e0, CBASE_Q * CH)],
                        col_v.at[pl.ds(0, CBASE_Q * CH)], ss[0])
                    b = pltpu.async_copy(
                        adj_hbm.at[i, 0, pl.ds(e0, CBASE_Q * CH)],
                        row_v.at[pl.ds(0, CBASE_Q * CH)], ss[0])
                    d = pltpu.async_copy(
                        val_hbm.at[i, pl.ds(e0, CBASE_Q * CH)],
                        val_v.at[pl.ds(0, CBASE_Q * CH)], ss[0])
                    a.wait(); b.wait(); d.wait()

                for t in range(PD):
                    prep_chunk(t, t, i)
                    start_gather(t, t)

                def slot(t, b, wait_sc, guard):
                    wait_gather(b)
                    bp = (b + PD) % NB

                    def prefetch():
                        if wait_sc:
                            wait_scatter(bp)
                        prep_chunk(t + PD, bp, i)
                        start_gather(t + PD, bp)
                    if guard:
                        pl.when(t + PD < cnt)(prefetch)
                    else:
                        prefetch()
                    mul(t, b)
                    start_scatter(b)

                for t in range(PD):
                    slot(t, t, False, False)
                for t in range(PD, NB):
                    slot(t, t, True, False)

                def sextet(tp, _t):
                    t0 = NB * tp
                    for b in range(NB):
                        slot(t0 + b, b, True, True)
                    return 0
                lax.fori_loop(1, CBASE_Q // NB, sextet, 0)

                @pl.when(s < CEXTRA)
                def _():
                    bx = CBASE_Q % NB
                    wait_gather(bx)
                    mul(CBASE_Q, bx)
                    start_scatter(bx)

                for b in range(NB):
                    wait_scatter(b)
                plsc.subcore_barrier()
                writeback(i)
                zero_acc()
                plsc.subcore_barrier()
            return 0

        lax.fori_loop(0, KMAX, adj_body, 0)

    return k(adj_idx, adj_vals, sup_units)


def kernel(x, adj_indices, adj_values, W1, b1, W2, b2):
    N, F = x.shape
    D, _, E = adj_indices.shape
    H = W1.shape[2]
    C = W2.shape[2]

    adj_idx = adj_indices.astype(jnp.int32)

    w1cat = jnp.zeros((F, PADW), jnp.float32)
    w1cat = w1cat.at[:, :D * H].set(
        jnp.transpose(W1, (1, 0, 2)).reshape(F, D * H))
    w2cat = jnp.zeros((PADW, PADW), jnp.float32)
    w2cat = w2cat.at[:D * H, :D * C].set(
        jnp.transpose(W2, (1, 0, 2)).reshape(D * H, D * C))
    b1pad = jnp.zeros((1, PADW), jnp.float32).at[0, :D * H].set(
        b1.reshape(D * H))
    b2pad = jnp.zeros((1, PADW), jnp.float32).at[0, :D * C].set(
        b2.reshape(D * C))

    sup1 = _tc_matmul(x, w1cat)
    h = _sc_spmm(adj_idx, adj_values,
                 sup1.reshape(N * (PADW // H), H), D, N, E, H)
    sup2 = _tc_matmul(h, w2cat, relu_bias=b1pad)
    h2 = _sc_spmm(adj_idx, adj_values,
                  sup2.reshape(N * (PADW // C), C), D, N, E, C)
    return _tc_pool(h2, b2pad, D, C)


# speedup vs baseline: 33.9565x; 1.1492x over previous
import functools

import jax
import jax.numpy as jnp
from jax import lax
from jax.experimental import pallas as pl
from jax.experimental.pallas import tpu as pltpu
from jax.experimental.pallas import tpu_sc as plsc

NC = 2
NS = 16
L = 16

BN = 2000
PADW = 512


def _tc_matmul(xin, w, relu_bias=None):
    N, K = xin.shape

    if relu_bias is None:
        def body(x_ref, w_ref, o_ref):
            o_ref[...] = jnp.dot(x_ref[...], w_ref[...],
                                 preferred_element_type=jnp.float32)
        args = (xin, w)
        in_specs = [
            pl.BlockSpec((BN, K), lambda jn: (jn, 0)),
            pl.BlockSpec((K, PADW), lambda jn: (0, 0)),
        ]
    else:
        def body(x_ref, b_ref, w_ref, o_ref):
            a = jnp.maximum(x_ref[...] + b_ref[...], 0.0)
            o_ref[...] = jnp.dot(a, w_ref[...],
                                 preferred_element_type=jnp.float32)
        args = (xin, relu_bias, w)
        in_specs = [
            pl.BlockSpec((BN, K), lambda jn: (jn, 0)),
            pl.BlockSpec((1, K), lambda jn: (0, 0)),
            pl.BlockSpec((K, PADW), lambda jn: (0, 0)),
        ]

    return pl.pallas_call(
        body,
        grid=(N // BN,),
        in_specs=in_specs,
        out_specs=pl.BlockSpec((BN, PADW), lambda jn: (jn, 0)),
        out_shape=jax.ShapeDtypeStruct((N, PADW), jnp.float32),
    )(*args)


def _tc_pool(h2, b2pad, D, C):
    N, _ = h2.shape

    def body(a_ref, b_ref, o_ref):
        a = jnp.maximum(a_ref[...] + b_ref[...], 0.0)
        m = a[:, 0:C]
        for i in range(1, D):
            m = jnp.maximum(m, a[:, i * C:(i + 1) * C])
        o_ref[...] = m

    return pl.pallas_call(
        body,
        grid=(N // BN,),
        in_specs=[
            pl.BlockSpec((BN, PADW), lambda jn: (jn, 0)),
            pl.BlockSpec((1, PADW), lambda jn: (0, 0)),
        ],
        out_specs=pl.BlockSpec((BN, C), lambda jn: (jn, 0)),
        out_shape=jax.ShapeDtypeStruct((N, C), jnp.float32),
    )(h2, b2pad)


def _sc_spmm(adj_idx, adj_vals, sup_units, D, N, E, W):
    CH = 128
    SR = 3
    CH2 = SR * CH
    NCHUNK = E // CH
    CBASE_Q = NCHUNK // NS
    CEXTRA = NCHUNK - CBASE_Q * NS
    NSLOT = (CBASE_Q * CH) // CH2
    KMAX = (D + NC - 1) // NC
    ZR = ((N // NS) + 7) // 8 * 8
    ZLAST = N - ZR * (NS - 1)
    NB = 4
    PD = 2
    UPR = PADW // W
    EPT_MAX = (CBASE_Q + 1) * CH

    mesh = plsc.VectorSubcoreMesh(core_axis_name="c", subcore_axis_name="s")

    scratch = [
        pltpu.VMEM((EPT_MAX,), jnp.int32),
        pltpu.VMEM((EPT_MAX,), jnp.int32),
        pltpu.VMEM((EPT_MAX,), jnp.float32),
        [pltpu.VMEM((SR, CH), jnp.int32) for _ in range(NB)],
        [pltpu.VMEM((SR, CH), jnp.int32) for _ in range(NB)],
        [pltpu.VMEM((CH2, W), jnp.float32) for _ in range(NB)],
        pltpu.VMEM((ZR, W), jnp.float32),
        pltpu.VMEM_SHARED((N, W), jnp.float32),
        [pltpu.SemaphoreType.DMA for _ in range(NB)],
        [pltpu.SemaphoreType.DMA for _ in range(NB)],
    ]

    @functools.partial(
        pl.kernel,
        out_type=jax.ShapeDtypeStruct((N, PADW), jnp.float32),
        mesh=mesh,
        scratch_types=scratch,
        compiler_params=pltpu.CompilerParams(use_tc_tiling_on_sc=False),
    )
    def k(adj_hbm, val_hbm, sup_hbm, out_hbm,
          col_v, row_v, val_v, cst, rst, gath, zero_v, acc_sh, sg, ss):
        c = lax.axis_index("c")
        s = lax.axis_index("s")

        def zb(j, _):
            zero_v[j, :] = jnp.zeros((W,), jnp.float32)
            return 0
        lax.fori_loop(0, ZR, zb, 0)

        def zero_acc():
            @pl.when(s < NS - 1)
            def _():
                pltpu.sync_copy(zero_v, acc_sh.at[pl.ds(s * ZR, ZR), :])

            @pl.when(s == NS - 1)
            def _():
                pltpu.sync_copy(zero_v.at[pl.ds(0, ZLAST), :],
                                acc_sh.at[pl.ds((NS - 1) * ZR, ZLAST), :])

        def out_block(co, width_src):
            @pl.when(s < NS - 1)
            def _():
                pltpu.sync_copy(width_src[0],
                                out_hbm.at[pl.ds(s * ZR, ZR), pl.ds(co, W)])

            @pl.when(s == NS - 1)
            def _():
                pltpu.sync_copy(
                    width_src[1],
                    out_hbm.at[pl.ds((NS - 1) * ZR, ZLAST), pl.ds(co, W)])

        def writeback(i):
            out_block(i * W, (acc_sh.at[pl.ds(s * ZR, ZR), :],
                              acc_sh.at[pl.ds((NS - 1) * ZR, ZLAST), :]))

        for j in range(D, UPR):
            out_block(j * W, (zero_v, zero_v.at[pl.ds(0, ZLAST), :]))

        cbase = s * CBASE_Q + jnp.minimum(s, CEXTRA)
        cnt = jnp.where(s < CEXTRA, CBASE_Q + 1, CBASE_Q)

        def prep_slot(t, b, i):
            for r in range(SR):
                for g in range(CH // L):
                    sl = pl.ds(t * CH2 + r * CH + g * L, L)
                    gsl = pl.ds(g * L, L)
                    cst[b][r, gsl] = col_v[sl] * UPR + i
                    rst[b][r, gsl] = row_v[sl]

        def start_gather(b):
            for r in range(SR):
                pltpu.async_copy(sup_hbm.at[cst[b].at[r]],
                                 gath[b].at[pl.ds(r * CH, CH)], sg[b])

        def wait_gather(b):
            pltpu.make_async_copy(sup_hbm.at[pl.ds(0, CH2), :], gath[b],
                                  sg[b]).wait()

        def start_scatter(b):
            for r in range(SR):
                pltpu.async_copy(gath[b].at[pl.ds(r * CH, CH)],
                                 acc_sh.at[rst[b].at[r]], ss[b], add=True)

        def wait_scatter(b):
            pltpu.make_async_copy(gath[b], acc_sh.at[pl.ds(0, CH2), :],
                                  ss[b]).wait()

        def mul(t, b):
            def mul_g(g, _):
                v16 = val_v[pl.ds(t * CH2 + g * L, L)]
                base = g * L
                for lane in range(L):
                    splat = jnp.broadcast_to(v16[lane], (L,))
                    gath[b][base + lane, :] = gath[b][base + lane, :] * splat
                return 0
            lax.fori_loop(0, CH2 // L, mul_g, 0, unroll=2)

        zero_acc()
        plsc.subcore_barrier()

        def adj_body(kk, _):
            i = kk * NC + c

            @pl.when(i < D)
            def _():
                e0 = cbase * CH

                @pl.when(s < CEXTRA)
                def _():
                    a = pltpu.async_copy(
                        adj_hbm.at[i, 1, pl.ds(e0, EPT_MAX)],
                        col_v, ss[0])
                    b = pltpu.async_copy(
                        adj_hbm.at[i, 0, pl.ds(e0, EPT_MAX)],
                        row_v, ss[0])
                    d = pltpu.async_copy(
                        val_hbm.at[i, pl.ds(e0, EPT_MAX)],
                        val_v, ss[0])
                    a.wait(); b.wait(); d.wait()

                @pl.when(s >= CEXTRA)
                def _():
                    a = pltpu.async_copy(
                        adj_hbm.at[i, 1, pl.ds(e0, CBASE_Q * CH)],
                        col_v.at[pl.ds(0, CBASE_Q * CH)], ss[0])
                    b = pltpu.async_copy(
                        adj_hbm.at[i, 0, pl.ds(e0, CBASE_Q * CH)],
                        row_v.at[pl.ds(0, CBASE_Q * CH)], ss[0])
                    d = pltpu.async_copy(
                        val_hbm.at[i, pl.ds(e0, CBASE_Q * CH)],
                        val_v.at[pl.ds(0, CBASE_Q * CH)], ss[0])
                    a.wait(); b.wait(); d.wait()

                for t in range(PD):
                    prep_slot(t, t, i)
                    start_gather(t)

                def slot(t, b, wait_sc, guard):
                    wait_gather(b)
                    bp = (b + PD) % NB

                    def prefetch():
                        if wait_sc:
                            wait_scatter(bp)
                        prep_slot(t + PD, bp, i)
                        start_gather(bp)
                    if guard:
                        pl.when(t + PD < NSLOT)(prefetch)
                    else:
                        prefetch()
                    mul(t, b)
                    start_scatter(b)

                for t in range(PD):
                    slot(t, t, False, False)
                for t in range(PD, NB):
                    slot(t, t, True, False)

                def quad(tp, _t):
                    t0 = NB * tp
                    for b in range(NB):
                        slot(t0 + b, b, True, True)
                    return 0
                lax.fori_loop(1, NSLOT // NB, quad, 0)

                for b in range(NB):
                    wait_scatter(b)

                @pl.when(s < CEXTRA)
                def _():
                    for g in range(CH // L):
                        sl = pl.ds(NSLOT * CH2 + g * L, L)
                        gsl = pl.ds(g * L, L)
                        cst[0][0, gsl] = col_v[sl] * UPR + i
                        rst[0][0, gsl] = row_v[sl]
                    pltpu.async_copy(sup_hbm.at[cst[0].at[0]],
                                     gath[0].at[pl.ds(0, CH)], sg[0]).wait()

                    def mul_t(g, _):
                        v16 = val_v[pl.ds(NSLOT * CH2 + g * L, L)]
                        base = g * L
                        for lane in range(L):
                            splat = jnp.broadcast_to(v16[lane], (L,))
                            gath[0][base + lane, :] = (
                                gath[0][base + lane, :] * splat)
                        return 0
                    lax.fori_loop(0, CH // L, mul_t, 0, unroll=2)
                    pltpu.async_copy(gath[0].at[pl.ds(0, CH)],
                                     acc_sh.at[rst[0].at[0]], ss[0],
                                     add=True).wait()
                plsc.subcore_barrier()
                writeback(i)
                zero_acc()
                plsc.subcore_barrier()
            return 0

        lax.fori_loop(0, KMAX, adj_body, 0)

    return k(adj_idx, adj_vals, sup_units)


def kernel(x, adj_indices, adj_values, W1, b1, W2, b2):
    N, F = x.shape
    D, _, E = adj_indices.shape
    H = W1.shape[2]
    C = W2.shape[2]

    adj_idx = adj_indices.astype(jnp.int32)

    w1cat = jnp.zeros((F, PADW), jnp.float32)
    w1cat = w1cat.at[:, :D * H].set(
        jnp.transpose(W1, (1, 0, 2)).reshape(F, D * H))
    w2cat = jnp.zeros((PADW, PADW), jnp.float32)
    w2cat = w2cat.at[:D * H, :D * C].set(
        jnp.transpose(W2, (1, 0, 2)).reshape(D * H, D * C))
    b1pad = jnp.zeros((1, PADW), jnp.float32).at[0, :D * H].set(
        b1.reshape(D * H))
    b2pad = jnp.zeros((1, PADW), jnp.float32).at[0, :D * C].set(
        b2.reshape(D * C))

    sup1 = _tc_matmul(x, w1cat)
    h = _sc_spmm(adj_idx, adj_values,
                 sup1.reshape(N * (PADW // H), H), D, N, E, H)
    sup2 = _tc_matmul(h, w2cat, relu_bias=b1pad)
    h2 = _sc_spmm(adj_idx, adj_values,
                  sup2.reshape(N * (PADW // C), C), D, N, E, C)
    return _tc_pool(h2, b2pad, D, C)


# speedup vs baseline: 37.1433x; 1.0939x over previous
import functools

import jax
import jax.numpy as jnp
from jax import lax
from jax.experimental import pallas as pl
from jax.experimental.pallas import tpu as pltpu
from jax.experimental.pallas import tpu_sc as plsc

NC = 2
NS = 16
L = 16

BN = 2000
PADW = 512


def _tc_matmul(xin, w, relu_bias=None):
    N, K = xin.shape

    if relu_bias is None:
        def body(x_ref, w_ref, o_ref):
            o_ref[...] = jnp.dot(x_ref[...], w_ref[...],
                                 preferred_element_type=jnp.float32)
        args = (xin, w)
        in_specs = [
            pl.BlockSpec((BN, K), lambda jn: (jn, 0)),
            pl.BlockSpec((K, PADW), lambda jn: (0, 0)),
        ]
    else:
        def body(x_ref, b_ref, w_ref, o_ref):
            a = jnp.maximum(x_ref[...] + b_ref[...], 0.0)
            o_ref[...] = jnp.dot(a, w_ref[...],
                                 preferred_element_type=jnp.float32)
        args = (xin, relu_bias, w)
        in_specs = [
            pl.BlockSpec((BN, K), lambda jn: (jn, 0)),
            pl.BlockSpec((1, K), lambda jn: (0, 0)),
            pl.BlockSpec((K, PADW), lambda jn: (0, 0)),
        ]

    return pl.pallas_call(
        body,
        grid=(N // BN,),
        in_specs=in_specs,
        out_specs=pl.BlockSpec((BN, PADW), lambda jn: (jn, 0)),
        out_shape=jax.ShapeDtypeStruct((N, PADW), jnp.float32),
    )(*args)


def _tc_pool(h2, b2pad, D, C):
    N, _ = h2.shape

    def body(a_ref, b_ref, o_ref):
        a = jnp.maximum(a_ref[...] + b_ref[...], 0.0)
        m = a[:, 0:C]
        for i in range(1, D):
            m = jnp.maximum(m, a[:, i * C:(i + 1) * C])
        o_ref[...] = m

    return pl.pallas_call(
        body,
        grid=(N // BN,),
        in_specs=[
            pl.BlockSpec((BN, PADW), lambda jn: (jn, 0)),
            pl.BlockSpec((1, PADW), lambda jn: (0, 0)),
        ],
        out_specs=pl.BlockSpec((BN, C), lambda jn: (jn, 0)),
        out_shape=jax.ShapeDtypeStruct((N, C), jnp.float32),
    )(h2, b2pad)


def _sc_spmm(adj_idx, adj_vals, sup_units, D, N, E, W):
    CH = 128
    SR = 3
    CH2 = SR * CH
    NCHUNK = E // CH
    CBASE_Q = NCHUNK // NS
    CEXTRA = NCHUNK - CBASE_Q * NS
    NSLOT = (CBASE_Q * CH) // CH2
    KMAX = (D + NC - 1) // NC
    ZR = ((N // NS) + 7) // 8 * 8
    ZLAST = N - ZR * (NS - 1)
    NB = 4
    PD = 2
    UPR = PADW // W
    EPT_MAX = (CBASE_Q + 1) * CH

    mesh = plsc.VectorSubcoreMesh(core_axis_name="c", subcore_axis_name="s")

    scratch = [
        pltpu.VMEM((CBASE_Q + 1, CH), jnp.int32),
        pltpu.VMEM((CBASE_Q + 1, CH), jnp.int32),
        pltpu.VMEM((EPT_MAX,), jnp.float32),
        [pltpu.VMEM((SR, CH), jnp.int32) for _ in range(NB)],
        [pltpu.VMEM((SR, CH), jnp.int32) for _ in range(NB)],
        [pltpu.VMEM((CH2, W), jnp.float32) for _ in range(NB)],
        pltpu.VMEM((ZR, W), jnp.float32),
        pltpu.VMEM_SHARED((N, W), jnp.float32),
        [pltpu.SemaphoreType.DMA for _ in range(NB)],
        [pltpu.SemaphoreType.DMA for _ in range(NB)],
    ]

    @functools.partial(
        pl.kernel,
        out_type=jax.ShapeDtypeStruct((N, PADW), jnp.float32),
        mesh=mesh,
        scratch_types=scratch,
        compiler_params=pltpu.CompilerParams(use_tc_tiling_on_sc=False),
    )
    def k(adj_hbm, val_hbm, sup_hbm, out_hbm,
          col_v, row_v, val_v, cst, rst, gath, zero_v, acc_sh, sg, ss):
        c = lax.axis_index("c")
        s = lax.axis_index("s")

        def zb(j, _):
            zero_v[j, :] = jnp.zeros((W,), jnp.float32)
            return 0
        lax.fori_loop(0, ZR, zb, 0)

        def zero_acc():
            @pl.when(s < NS - 1)
            def _():
                pltpu.sync_copy(zero_v, acc_sh.at[pl.ds(s * ZR, ZR), :])

            @pl.when(s == NS - 1)
            def _():
                pltpu.sync_copy(zero_v.at[pl.ds(0, ZLAST), :],
                                acc_sh.at[pl.ds((NS - 1) * ZR, ZLAST), :])

        def out_block(co, width_src):
            @pl.when(s < NS - 1)
            def _():
                pltpu.sync_copy(width_src[0],
                                out_hbm.at[pl.ds(s * ZR, ZR), pl.ds(co, W)])

            @pl.when(s == NS - 1)
            def _():
                pltpu.sync_copy(
                    width_src[1],
                    out_hbm.at[pl.ds((NS - 1) * ZR, ZLAST), pl.ds(co, W)])

        def writeback(i):
            out_block(i * W, (acc_sh.at[pl.ds(s * ZR, ZR), :],
                              acc_sh.at[pl.ds((NS - 1) * ZR, ZLAST), :]))

        for j in range(D, UPR):
            out_block(j * W, (zero_v, zero_v.at[pl.ds(0, ZLAST), :]))

        cbase = s * CBASE_Q + jnp.minimum(s, CEXTRA)
        cnt = jnp.where(s < CEXTRA, CBASE_Q + 1, CBASE_Q)

        def prep_slot(t, b, i):
            for r in range(SR):
                cc = t * SR + r
                for g in range(CH // L):
                    gsl = pl.ds(g * L, L)
                    cst[b][r, gsl] = col_v[cc, gsl] * UPR + i
                    rst[b][r, gsl] = row_v[cc, gsl]

        def start_gather(b):
            for r in range(SR):
                pltpu.async_copy(sup_hbm.at[cst[b].at[r]],
                                 gath[b].at[pl.ds(r * CH, CH)], sg[b])

        def wait_gather(b):
            pltpu.make_async_copy(sup_hbm.at[pl.ds(0, CH2), :], gath[b],
                                  sg[b]).wait()

        def start_scatter(b):
            for r in range(SR):
                pltpu.async_copy(gath[b].at[pl.ds(r * CH, CH)],
                                 acc_sh.at[rst[b].at[r]], ss[b], add=True)

        def wait_scatter(b):
            pltpu.make_async_copy(gath[b], acc_sh.at[pl.ds(0, CH2), :],
                                  ss[b]).wait()

        def mul(t, b):
            def mul_g(g, _):
                v16 = val_v[pl.ds(t * CH2 + g * L, L)]
                base = g * L
                for lane in range(L):
                    splat = jnp.broadcast_to(v16[lane], (L,))
                    gath[b][base + lane, :] = gath[b][base + lane, :] * splat
                return 0
            lax.fori_loop(0, CH2 // L, mul_g, 0, unroll=2)

        zero_acc()
        plsc.subcore_barrier()

        def adj_body(kk, _):
            i = kk * NC + c

            @pl.when(i < D)
            def _():
                e0 = cbase * CH

                @pl.when(s < CEXTRA)
                def _():
                    a = pltpu.async_copy(
                        adj_hbm.at[i, pl.ds(cbase, CBASE_Q + 1), 1, :],
                        col_v, ss[0])
                    b = pltpu.async_copy(
                        adj_hbm.at[i, pl.ds(cbase, CBASE_Q + 1), 0, :],
                        row_v, ss[0])
                    d = pltpu.async_copy(
                        val_hbm.at[i, pl.ds(e0, EPT_MAX)],
                        val_v, ss[0])
                    a.wait(); b.wait(); d.wait()

                @pl.when(s >= CEXTRA)
                def _():
                    a = pltpu.async_copy(
                        adj_hbm.at[i, pl.ds(cbase, CBASE_Q), 1, :],
                        col_v.at[pl.ds(0, CBASE_Q), :], ss[0])
                    b = pltpu.async_copy(
                        adj_hbm.at[i, pl.ds(cbase, CBASE_Q), 0, :],
                        row_v.at[pl.ds(0, CBASE_Q), :], ss[0])
                    d = pltpu.async_copy(
                        val_hbm.at[i, pl.ds(e0, CBASE_Q * CH)],
                        val_v.at[pl.ds(0, CBASE_Q * CH)], ss[0])
                    a.wait(); b.wait(); d.wait()

                for t in range(PD):
                    prep_slot(t, t, i)
                    start_gather(t)

                def slot(t, b, wait_sc, guard):
                    wait_gather(b)
                    bp = (b + PD) % NB

                    def prefetch():
                        if wait_sc:
                            wait_scatter(bp)
                        prep_slot(t + PD, bp, i)
                        start_gather(bp)
                    if guard:
                        pl.when(t + PD < NSLOT)(prefetch)
                    else:
                        prefetch()
                    mul(t, b)
                    start_scatter(b)

                for t in range(PD):
                    slot(t, t, False, False)
                for t in range(PD, NB):
                    slot(t, t, True, False)

                def quad(tp, _t):
                    t0 = NB * tp
                    for b in range(NB):
                        slot(t0 + b, b, True, True)
                    return 0
                lax.fori_loop(1, NSLOT // NB, quad, 0)

                for b in range(NB):
                    wait_scatter(b)

                @pl.when(s < CEXTRA)
                def _():
                    for g in range(CH // L):
                        gsl = pl.ds(g * L, L)
                        cst[0][0, gsl] = col_v[CBASE_Q, gsl] * UPR + i
                        rst[0][0, gsl] = row_v[CBASE_Q, gsl]
                    pltpu.async_copy(sup_hbm.at[cst[0].at[0]],
                                     gath[0].at[pl.ds(0, CH)], sg[0]).wait()

                    def mul_t(g, _):
                        v16 = val_v[pl.ds(NSLOT * CH2 + g * L, L)]
                        base = g * L
                        for lane in range(L):
                            splat = jnp.broadcast_to(v16[lane], (L,))
                            gath[0][base + lane, :] = (
                                gath[0][base + lane, :] * splat)
                        return 0
                    lax.fori_loop(0, CH // L, mul_t, 0, unroll=2)
                    pltpu.async_copy(gath[0].at[pl.ds(0, CH)],
                                     acc_sh.at[rst[0].at[0]], ss[0],
                                     add=True).wait()
                plsc.subcore_barrier()
                writeback(i)
                zero_acc()
                plsc.subcore_barrier()
            return 0

        lax.fori_loop(0, KMAX, adj_body, 0)

    return k(adj_idx, adj_vals, sup_units)


def kernel(x, adj_indices, adj_values, W1, b1, W2, b2):
    N, F = x.shape
    D, _, E = adj_indices.shape
    H = W1.shape[2]
    C = W2.shape[2]

    adj_idx = adj_indices.astype(jnp.int32).reshape(D, 2, E)
    adj_idx = jnp.swapaxes(adj_idx.reshape(D, 2, E // 128, 128), 1, 2)

    w1cat = jnp.zeros((F, PADW), jnp.float32)
    w1cat = w1cat.at[:, :D * H].set(
        jnp.transpose(W1, (1, 0, 2)).reshape(F, D * H))
    w2cat = jnp.zeros((PADW, PADW), jnp.float32)
    w2cat = w2cat.at[:D * H, :D * C].set(
        jnp.transpose(W2, (1, 0, 2)).reshape(D * H, D * C))
    b1pad = jnp.zeros((1, PADW), jnp.float32).at[0, :D * H].set(
        b1.reshape(D * H))
    b2pad = jnp.zeros((1, PADW), jnp.float32).at[0, :D * C].set(
        b2.reshape(D * C))

    sup1 = _tc_matmul(x, w1cat)
    h = _sc_spmm(adj_idx, adj_values,
                 sup1.reshape(N * (PADW // H), H), D, N, E, H)
    sup2 = _tc_matmul(h, w2cat, relu_bias=b1pad)
    h2 = _sc_spmm(adj_idx, adj_values,
                  sup2.reshape(N * (PADW // C), C), D, N, E, C)
    return _tc_pool(h2, b2pad, D, C)


# speedup vs baseline: 40.1675x; 1.0814x over previous
import functools

import jax
import jax.numpy as jnp
from jax import lax
from jax.experimental import pallas as pl
from jax.experimental.pallas import tpu as pltpu
from jax.experimental.pallas import tpu_sc as plsc

NC = 2
NS = 16
L = 16

BN = 2000
PADW = 512


def _tc_matmul(xin, w, relu_bias=None):
    N, K = xin.shape

    if relu_bias is None:
        def body(x_ref, w_ref, o_ref):
            o_ref[...] = jnp.dot(x_ref[...], w_ref[...],
                                 preferred_element_type=jnp.float32)
        args = (xin, w)
        in_specs = [
            pl.BlockSpec((BN, K), lambda jn: (jn, 0)),
            pl.BlockSpec((K, PADW), lambda jn: (0, 0)),
        ]
    else:
        def body(x_ref, b_ref, w_ref, o_ref):
            a = jnp.maximum(x_ref[...] + b_ref[...], 0.0)
            o_ref[...] = jnp.dot(a, w_ref[...],
                                 preferred_element_type=jnp.float32)
        args = (xin, relu_bias, w)
        in_specs = [
            pl.BlockSpec((BN, K), lambda jn: (jn, 0)),
            pl.BlockSpec((1, K), lambda jn: (0, 0)),
            pl.BlockSpec((K, PADW), lambda jn: (0, 0)),
        ]

    return pl.pallas_call(
        body,
        grid=(N // BN,),
        in_specs=in_specs,
        out_specs=pl.BlockSpec((BN, PADW), lambda jn: (jn, 0)),
        out_shape=jax.ShapeDtypeStruct((N, PADW), jnp.float32),
    )(*args)


def _tc_pool(h2, b2pad, D, C):
    N, _ = h2.shape

    def body(a_ref, b_ref, o_ref):
        a = jnp.maximum(a_ref[...] + b_ref[...], 0.0)
        m = a[:, 0:C]
        for i in range(1, D):
            m = jnp.maximum(m, a[:, i * C:(i + 1) * C])
        o_ref[...] = m

    return pl.pallas_call(
        body,
        grid=(N // BN,),
        in_specs=[
            pl.BlockSpec((BN, PADW), lambda jn: (jn, 0)),
            pl.BlockSpec((1, PADW), lambda jn: (0, 0)),
        ],
        out_specs=pl.BlockSpec((BN, C), lambda jn: (jn, 0)),
        out_shape=jax.ShapeDtypeStruct((N, C), jnp.float32),
    )(h2, b2pad)


def _sc_spmm(adj_idx, adj_vals, sup_units, D, N, E, W):
    CH = 128
    SR = 2
    CH2 = SR * CH
    NCHUNK = E // CH
    CBASE_Q = NCHUNK // NS
    CEXTRA = NCHUNK - CBASE_Q * NS
    NSLOT = (CBASE_Q * CH) // CH2
    KMAX = (D + NC - 1) // NC
    ZR = ((N // NS) + 7) // 8 * 8
    ZLAST = N - ZR * (NS - 1)
    NB = 6
    PD = 3
    UPR = PADW // W
    EPT_MAX = (CBASE_Q + 1) * CH

    mesh = plsc.VectorSubcoreMesh(core_axis_name="c", subcore_axis_name="s")

    scratch = [
        pltpu.VMEM((CBASE_Q + 1, CH), jnp.int32),
        pltpu.VMEM((CBASE_Q + 1, CH), jnp.int32),
        pltpu.VMEM((EPT_MAX,), jnp.float32),
        [pltpu.VMEM((SR, CH), jnp.int32) for _ in range(NB)],
        [pltpu.VMEM((SR, CH), jnp.int32) for _ in range(NB)],
        [pltpu.VMEM((CH2, W), jnp.float32) for _ in range(NB)],
        pltpu.VMEM((ZR, W), jnp.float32),
        pltpu.VMEM_SHARED((N, W), jnp.float32),
        [pltpu.SemaphoreType.DMA for _ in range(NB)],
        [pltpu.SemaphoreType.DMA for _ in range(NB)],
    ]

    @functools.partial(
        pl.kernel,
        out_type=jax.ShapeDtypeStruct((N, PADW), jnp.float32),
        mesh=mesh,
        scratch_types=scratch,
        compiler_params=pltpu.CompilerParams(use_tc_tiling_on_sc=False),
    )
    def k(adj_hbm, val_hbm, sup_hbm, out_hbm,
          col_v, row_v, val_v, cst, rst, gath, zero_v, acc_sh, sg, ss):
        c = lax.axis_index("c")
        s = lax.axis_index("s")

        def zb(j, _):
            zero_v[j, :] = jnp.zeros((W,), jnp.float32)
            return 0
        lax.fori_loop(0, ZR, zb, 0)

        def zero_acc():
            @pl.when(s < NS - 1)
            def _():
                pltpu.sync_copy(zero_v, acc_sh.at[pl.ds(s * ZR, ZR), :])

            @pl.when(s == NS - 1)
            def _():
                pltpu.sync_copy(zero_v.at[pl.ds(0, ZLAST), :],
                                acc_sh.at[pl.ds((NS - 1) * ZR, ZLAST), :])

        def out_block(co, width_src):
            @pl.when(s < NS - 1)
            def _():
                pltpu.sync_copy(width_src[0],
                                out_hbm.at[pl.ds(s * ZR, ZR), pl.ds(co, W)])

            @pl.when(s == NS - 1)
            def _():
                pltpu.sync_copy(
                    width_src[1],
                    out_hbm.at[pl.ds((NS - 1) * ZR, ZLAST), pl.ds(co, W)])

        def writeback(i):
            out_block(i * W, (acc_sh.at[pl.ds(s * ZR, ZR), :],
                              acc_sh.at[pl.ds((NS - 1) * ZR, ZLAST), :]))

        for j in range(D, UPR):
            out_block(j * W, (zero_v, zero_v.at[pl.ds(0, ZLAST), :]))

        cbase = s * CBASE_Q + jnp.minimum(s, CEXTRA)
        cnt = jnp.where(s < CEXTRA, CBASE_Q + 1, CBASE_Q)

        def prep_slot(t, b, i):
            for r in range(SR):
                cc = t * SR + r
                for g in range(CH // L):
                    gsl = pl.ds(g * L, L)
                    cst[b][r, gsl] = col_v[cc, gsl] * UPR + i
                    rst[b][r, gsl] = row_v[cc, gsl]

        def start_gather(b):
            for r in range(SR):
                pltpu.async_copy(sup_hbm.at[cst[b].at[r]],
                                 gath[b].at[pl.ds(r * CH, CH)], sg[b])

        def wait_gather(b):
            pltpu.make_async_copy(sup_hbm.at[pl.ds(0, CH2), :], gath[b],
                                  sg[b]).wait()

        def start_scatter(b):
            for r in range(SR):
                pltpu.async_copy(gath[b].at[pl.ds(r * CH, CH)],
                                 acc_sh.at[rst[b].at[r]], ss[b], add=True)

        def wait_scatter(b):
            pltpu.make_async_copy(gath[b], acc_sh.at[pl.ds(0, CH2), :],
                                  ss[b]).wait()

        def mul(t, b):
            def mul_g(g, _):
                v16 = val_v[pl.ds(t * CH2 + g * L, L)]
                base = g * L
                for lane in range(L):
                    splat = jnp.broadcast_to(v16[lane], (L,))
                    gath[b][base + lane, :] = gath[b][base + lane, :] * splat
                return 0
            lax.fori_loop(0, CH2 // L, mul_g, 0, unroll=2)

        zero_acc()
        plsc.subcore_barrier()

        def adj_body(kk, _):
            i = kk * NC + c

            @pl.when(i < D)
            def _():
                e0 = cbase * CH

                @pl.when(s < CEXTRA)
                def _():
                    a = pltpu.async_copy(
                        adj_hbm.at[i, pl.ds(cbase, CBASE_Q + 1), 1, :],
                        col_v, ss[0])
                    b = pltpu.async_copy(
                        adj_hbm.at[i, pl.ds(cbase, CBASE_Q + 1), 0, :],
                        row_v, ss[0])
                    d = pltpu.async_copy(
                        val_hbm.at[i, pl.ds(e0, EPT_MAX)],
                        val_v, ss[0])
                    a.wait(); b.wait(); d.wait()

                @pl.when(s >= CEXTRA)
                def _():
                    a = pltpu.async_copy(
                        adj_hbm.at[i, pl.ds(cbase, CBASE_Q), 1, :],
                        col_v.at[pl.ds(0, CBASE_Q), :], ss[0])
                    b = pltpu.async_copy(
                        adj_hbm.at[i, pl.ds(cbase, CBASE_Q), 0, :],
                        row_v.at[pl.ds(0, CBASE_Q), :], ss[0])
                    d = pltpu.async_copy(
                        val_hbm.at[i, pl.ds(e0, CBASE_Q * CH)],
                        val_v.at[pl.ds(0, CBASE_Q * CH)], ss[0])
                    a.wait(); b.wait(); d.wait()

                for t in range(PD):
                    prep_slot(t, t, i)
                    start_gather(t)

                def slot(t, b, wait_sc, guard):
                    wait_gather(b)
                    bp = (b + PD) % NB

                    def prefetch():
                        if wait_sc:
                            wait_scatter(bp)
                        prep_slot(t + PD, bp, i)
                        start_gather(bp)
                    if guard:
                        pl.when(t + PD < NSLOT)(prefetch)
                    else:
                        prefetch()
                    mul(t, b)
                    start_scatter(b)

                for t in range(PD):
                    slot(t, t, False, False)
                for t in range(PD, NB):
                    slot(t, t, True, False)

                def quad(tp, _t):
                    t0 = NB * tp
                    for b in range(NB):
                        slot(t0 + b, b, True, True)
                    return 0
                lax.fori_loop(1, NSLOT // NB, quad, 0)

                for b in range(NB):
                    wait_scatter(b)

                @pl.when(s < CEXTRA)
                def _():
                    for g in range(CH // L):
                        gsl = pl.ds(g * L, L)
                        cst[0][0, gsl] = col_v[CBASE_Q, gsl] * UPR + i
                        rst[0][0, gsl] = row_v[CBASE_Q, gsl]
                    pltpu.async_copy(sup_hbm.at[cst[0].at[0]],
                                     gath[0].at[pl.ds(0, CH)], sg[0]).wait()

                    def mul_t(g, _):
                        v16 = val_v[pl.ds(NSLOT * CH2 + g * L, L)]
                        base = g * L
                        for lane in range(L):
                            splat = jnp.broadcast_to(v16[lane], (L,))
                            gath[0][base + lane, :] = (
                                gath[0][base + lane, :] * splat)
                        return 0
                    lax.fori_loop(0, CH // L, mul_t, 0, unroll=2)
                    pltpu.async_copy(gath[0].at[pl.ds(0, CH)],
                                     acc_sh.at[rst[0].at[0]], ss[0],
                                     add=True).wait()
                plsc.subcore_barrier()
                writeback(i)
                zero_acc()
                plsc.subcore_barrier()
            return 0

        lax.fori_loop(0, KMAX, adj_body, 0)

    return k(adj_idx, adj_vals, sup_units)


def kernel(x, adj_indices, adj_values, W1, b1, W2, b2):
    N, F = x.shape
    D, _, E = adj_indices.shape
    H = W1.shape[2]
    C = W2.shape[2]

    adj_idx = adj_indices.astype(jnp.int32).reshape(D, 2, E)
    adj_idx = jnp.swapaxes(adj_idx.reshape(D, 2, E // 128, 128), 1, 2)

    w1cat = jnp.zeros((F, PADW), jnp.float32)
    w1cat = w1cat.at[:, :D * H].set(
        jnp.transpose(W1, (1, 0, 2)).reshape(F, D * H))
    w2cat = jnp.zeros((PADW, PADW), jnp.float32)
    w2cat = w2cat.at[:D * H, :D * C].set(
        jnp.transpose(W2, (1, 0, 2)).reshape(D * H, D * C))
    b1pad = jnp.zeros((1, PADW), jnp.float32).at[0, :D * H].set(
        b1.reshape(D * H))
    b2pad = jnp.zeros((1, PADW), jnp.float32).at[0, :D * C].set(
        b2.reshape(D * C))

    sup1 = _tc_matmul(x, w1cat)
    h = _sc_spmm(adj_idx, adj_values,
                 sup1.reshape(N * (PADW // H), H), D, N, E, H)
    sup2 = _tc_matmul(h, w2cat, relu_bias=b1pad)
    h2 = _sc_spmm(adj_idx, adj_values,
                  sup2.reshape(N * (PADW // C), C), D, N, E, C)
    return _tc_pool(h2, b2pad, D, C)
